# baseline (speedup 1.0000x reference)
import functools

import jax
import jax.numpy as jnp
from jax import lax
from jax.experimental import pallas as pl
from jax.experimental.pallas import tpu as pltpu
from jax.experimental.pallas import tpu_sc as plsc

_B, _T, _D = 4096, 200, 64
_N = _B * _T
_NW = 32
_NPW = _N // _NW
_CH = 128
_NCH = _NPW // _CH
_NP = _NCH // 2

_mesh = plsc.VectorSubcoreMesh(core_axis_name="c", subcore_axis_name="s")


@functools.partial(
    pl.kernel,
    out_type=jax.ShapeDtypeStruct((_T, 8, 32, 8, 128), jnp.float32),
    mesh=_mesh,
    scratch_types=[
        pltpu.VMEM((_NPW,), jnp.int32),
        pltpu.VMEM((2, _CH, _D), jnp.float32),
        pltpu.VMEM((2, _D, 128), jnp.float32),
        pltpu.VMEM_SHARED((8192, _D), jnp.float32),
        pltpu.SemaphoreType.DMA,
        pltpu.SemaphoreType.DMA,
    ],
    compiler_params=pltpu.CompilerParams(use_tc_tiling_on_sc=False,
                                         needs_layout_passes=False),
)
def _emb_gather(x_hbm, tab_hbm, out_hbm, idx_v, gbuf, sbuf, tab_sh, gsem, ssem):
    sid = lax.axis_index("s")
    wid = sid * 2 + lax.axis_index("c")
    pltpu.sync_copy(x_hbm.at[pl.ds(wid * _NPW, _NPW)], idx_v)
    _RPS = 8192 // 16
    pltpu.sync_copy(tab_hbm.at[pl.ds(sid * _RPS, _RPS), :],
                    tab_sh.at[pl.ds(sid * _RPS, _RPS), :])
    plsc.subcore_barrier()

    qbase = wid * _NCH

    def fire_gather(c, rb):
        pltpu.async_copy(
            tab_sh.at[idx_v.at[pl.ds(c * _CH, _CH)]], gbuf.at[rb], gsem)

    def wait_gather():
        pltpu.make_async_copy(
            tab_sh.at[idx_v.at[pl.ds(0, _CH)]], gbuf.at[0], gsem).wait()

    iota = lax.iota(jnp.int32, 16)
    biotas = [iota + (j * 16) for j in range(8)]

    def transpose(rb):
        @plsc.parallel_loop(0, _D, unroll=8)
        def tbody(i):
            dv = ((iota + i) & 15) + (i & ~15)
            for j in range(8):
                vals = plsc.load_gather(gbuf.at[rb], [biotas[j], dv])
                plsc.store_scatter(sbuf.at[rb], [dv, biotas[j]], vals)

    def fire_scatter(c, rb):
        q = qbase + c
        t = q // 32
        bb = q % 32
        for dt in range(8):
            pltpu.async_copy(
                sbuf.at[rb, pl.ds(dt * 8, 8), :],
                out_hbm.at[t, dt, bb], ssem)

    def drain_scatters():
        for _ in range(8):
            pltpu.make_async_copy(
                sbuf.at[0, pl.ds(0, 8), :],
                out_hbm.at[0, 0, 0], ssem).wait()

    fire_gather(0, 0)

    def pbody(p, carry):
        c0 = 2 * p
        @pl.when(p > 0)
        def _():
            drain_scatters()

        fire_gather(c0 + 1, 1)
        wait_gather()
        transpose(0)
        fire_scatter(c0, 0)

        @pl.when(p > 0)
        def _():
            drain_scatters()

        @pl.when(p + 1 < _NP)
        def _():
            fire_gather(c0 + 2, 0)

        wait_gather()
        transpose(1)
        fire_scatter(c0 + 1, 1)
        return carry

    lax.fori_loop(0, _NP, pbody, 0)
    drain_scatters()
    drain_scatters()


def kernel(x, table):
    xt = x.T.reshape(_N).astype(jnp.int32)
    out5 = _emb_gather(xt, table)
    return out5.transpose(2, 4, 0, 1, 3).reshape(_B, _T, _D)


# speedup vs baseline: 1.0682x; 1.0682x over previous
import functools

import jax
import jax.numpy as jnp
from jax import lax
from jax.experimental import pallas as pl
from jax.experimental.pallas import tpu as pltpu
from jax.experimental.pallas import tpu_sc as plsc

_B, _T, _D = 4096, 200, 64
_N = _B * _T
_NW = 32
_NPW = _N // _NW
_CH = 128
_NCH = _NPW // _CH
_NP = _NCH // 2

_mesh = plsc.VectorSubcoreMesh(core_axis_name="c", subcore_axis_name="s")


@functools.partial(
    pl.kernel,
    out_type=jax.ShapeDtypeStruct((_T, 8, 32, 8, 128), jnp.float32),
    mesh=_mesh,
    scratch_types=[
        pltpu.VMEM((_NPW,), jnp.int32),
        pltpu.VMEM((2, _CH, _D), jnp.float32),
        pltpu.VMEM((2, _D, 128), jnp.float32),
        pltpu.VMEM_SHARED((8192, _D), jnp.float32),
        pltpu.SemaphoreType.DMA,
        pltpu.SemaphoreType.DMA,
    ],
    compiler_params=pltpu.CompilerParams(use_tc_tiling_on_sc=False,
                                         needs_layout_passes=False),
)
def _emb_gather(x_hbm, tab_hbm, out_hbm, idx_v, gbuf, sbuf, tab_sh, gsem, ssem):
    sid = lax.axis_index("s")
    wid = sid * 2 + lax.axis_index("c")
    pltpu.sync_copy(x_hbm.at[pl.ds(wid * _NPW, _NPW)], idx_v)
    _RPS = 8192 // 16
    pltpu.sync_copy(tab_hbm.at[pl.ds(sid * _RPS, _RPS), :],
                    tab_sh.at[pl.ds(sid * _RPS, _RPS), :])
    plsc.subcore_barrier()

    qbase = wid * _NCH

    def fire_gather(c, rb):
        pltpu.async_copy(
            tab_sh.at[idx_v.at[pl.ds(c * _CH, _CH)]], gbuf.at[rb], gsem)

    def wait_gather():
        pltpu.make_async_copy(
            tab_sh.at[idx_v.at[pl.ds(0, _CH)]], gbuf.at[0], gsem).wait()

    iota = lax.iota(jnp.int32, 16)
    biotas = [iota + (j * 16) for j in range(8)]

    def transpose(rb):
        @plsc.parallel_loop(0, _D, unroll=2)
        def tbody(i):
            dv = ((iota + i) & 15) + (i & ~15)
            for j in range(8):
                vals = plsc.load_gather(gbuf.at[rb], [biotas[j], dv])
                plsc.store_scatter(sbuf.at[rb], [dv, biotas[j]], vals)

    def fire_scatter(c, rb):
        q = qbase + c
        t = q // 32
        bb = q % 32
        for dt in range(8):
            pltpu.async_copy(
                sbuf.at[rb, pl.ds(dt * 8, 8), :],
                out_hbm.at[t, dt, bb], ssem)

    def drain_scatters():
        for _ in range(8):
            pltpu.make_async_copy(
                sbuf.at[0, pl.ds(0, 8), :],
                out_hbm.at[0, 0, 0], ssem).wait()

    fire_gather(0, 0)

    def pbody(p, carry):
        c0 = 2 * p
        @pl.when(p > 0)
        def _():
            drain_scatters()

        fire_gather(c0 + 1, 1)
        wait_gather()
        transpose(0)
        fire_scatter(c0, 0)

        @pl.when(p > 0)
        def _():
            drain_scatters()

        @pl.when(p + 1 < _NP)
        def _():
            fire_gather(c0 + 2, 0)

        wait_gather()
        transpose(1)
        fire_scatter(c0 + 1, 1)
        return carry

    lax.fori_loop(0, _NP, pbody, 0)
    drain_scatters()
    drain_scatters()


def kernel(x, table):
    xt = x.T.reshape(_N).astype(jnp.int32)
    out5 = _emb_gather(xt, table)
    return out5.transpose(2, 4, 0, 1, 3).reshape(_B, _T, _D)
